# baseline (speedup 1.0000x reference)
import functools

import jax
import jax.numpy as jnp
from jax import lax
from jax.experimental import pallas as pl
from jax.experimental.pallas import tpu as pltpu
from jax.experimental.pallas import tpu_sc as plsc

_INFO = plsc.get_sparse_core_info()
_NC = _INFO.num_cores
_NS = _INFO.num_subcores
_NW = _NC * _NS

_CHUNK = 128


@functools.lru_cache(maxsize=None)
def _build(batch, dim):
    assert batch % _NW == 0
    b_per_w = batch // _NW
    assert b_per_w % _CHUNK == 0 or b_per_w <= _CHUNK
    n_chunks = max(1, b_per_w // _CHUNK)
    chunk = min(b_per_w, _CHUNK)

    mesh = plsc.VectorSubcoreMesh(core_axis_name="c", subcore_axis_name="s")

    @functools.partial(
        pl.kernel,
        mesh=mesh,
        out_type=jax.ShapeDtypeStruct((batch, dim), jnp.float32),
        scratch_types=[
            pltpu.VMEM((b_per_w,), jnp.int32),
            pltpu.VMEM((b_per_w, dim), jnp.float32),
            pltpu.SemaphoreType.DMA,
        ],
        compiler_params=pltpu.CompilerParams(use_tc_tiling_on_sc=False),
    )
    def gather_kernel(idx_hbm, table_hbm, out_hbm, idx_v, rows_v, sem):
        wid = lax.axis_index("s") * _NC + lax.axis_index("c")
        base = wid * b_per_w
        pltpu.sync_copy(idx_hbm.at[pl.ds(base, b_per_w)], idx_v)
        copies = []
        for c in range(n_chunks):
            copies.append(
                pltpu.async_copy(
                    table_hbm.at[idx_v.at[pl.ds(c * chunk, chunk)]],
                    rows_v.at[pl.ds(c * chunk, chunk)],
                    sem,
                )
            )
        for cp in copies:
            cp.wait()
        pltpu.sync_copy(rows_v, out_hbm.at[pl.ds(base, b_per_w)])

    return gather_kernel


def kernel(g, h, r, norm, W):
    idx = jnp.squeeze(h).astype(jnp.int32)
    return _build(idx.shape[0], W.shape[1])(idx, W)


# speedup vs baseline: 4.0513x; 4.0513x over previous
import functools

import jax
import jax.numpy as jnp
from jax import lax
from jax.experimental import pallas as pl
from jax.experimental.pallas import tpu as pltpu
from jax.experimental.pallas import tpu_sc as plsc

_INFO = plsc.get_sparse_core_info()
_NC = _INFO.num_cores
_NS = _INFO.num_subcores
_NW = _NC * _NS

_N = 1000000
_B = 16384
_D = 64
_CCOLS = 512
_NCHUNK = 1954
_TAIL_C = _NCHUNK - 1
_TAIL_LO = _N - 128
_JMAX = 62
_CAP = 2048
_DUMP = _B
_LANE = None


def _k(c):
    return jnp.int32(c)


@functools.lru_cache(maxsize=None)
def _build():
    mesh = plsc.VectorSubcoreMesh(core_axis_name="c", subcore_axis_name="s")

    @functools.partial(
        pl.kernel,
        mesh=mesh,
        out_type=jax.ShapeDtypeStruct((_B + 16, 128), jnp.float32),
        scratch_types=[
            pltpu.VMEM((_B,), jnp.int32),
            pltpu.VMEM((_B + 64,), jnp.int32),
            pltpu.VMEM((_B + 64,), jnp.int32),
            pltpu.VMEM((_CAP + 16,), jnp.int32),
            pltpu.VMEM((_CAP + 16,), jnp.int32),
            pltpu.VMEM((_D, _CCOLS), jnp.float32),
            pltpu.VMEM((_D, _CCOLS), jnp.float32),
            pltpu.VMEM((32, 128), jnp.float32),
            pltpu.VMEM((3, 16), jnp.int32),
            pltpu.SMEM((8,), jnp.int32),
            pltpu.SemaphoreType.DMA,
            pltpu.SemaphoreType.DMA,
            pltpu.SemaphoreType.DMA,
            pltpu.SemaphoreType.DMA,
        ],
        compiler_params=pltpu.CompilerParams(needs_layout_passes=False),
    )
    def body(idx_hbm, table_hbm, tail_hbm, out_hbm, idx_v, mval_v, mpos_v,
             cval_v, cpos_v, chunk_a, chunk_b, stage_v, spos_v, state_s,
             sem_ca, sem_cb, sem_sa, sem_sb):
        wid = lax.axis_index("s") * _NC + lax.axis_index("c")
        lane = lax.iota(jnp.int32, 16)
        dump16 = _DUMP + lane

        pltpu.sync_copy(idx_hbm, idx_v)

        def scan_body(q, m):
            v = idx_v[pl.ds(q * 16, 16)]
            mask = ((v >> 9) & (_NW - 1)) == wid
            bpos = q * 16 + lane
            plsc.store_compressed(mval_v.at[pl.ds(m, 16)], v, mask=mask)
            plsc.store_compressed(mpos_v.at[pl.ds(m, 16)], bpos, mask=mask)
            return m + jnp.sum(mask.astype(jnp.int32))

        m = lax.fori_loop(0, _B // 16, scan_body, jnp.int32(0))
        n_groups = (m + 15) >> 4

        def fire(c, buf, sem):
            @pl.when(c <= _TAIL_C - 1)
            def _():
                pltpu.async_copy(
                    table_hbm.at[:, pl.ds(c * _CCOLS, _CCOLS)], buf, sem)

            @pl.when(c == _TAIL_C)
            def _():
                pltpu.async_copy(tail_hbm, buf.at[:, pl.ds(0, 128)], sem)

        def wait(c, buf, sem):
            @pl.when(c <= _TAIL_C - 1)
            def _():
                pltpu.make_async_copy(
                    table_hbm.at[:, pl.ds(c * _CCOLS, _CCOLS)], buf, sem
                ).wait()

            @pl.when(c == _TAIL_C)
            def _():
                pltpu.make_async_copy(
                    tail_hbm, buf.at[:, pl.ds(0, 128)], sem).wait()

        def drain_a():
            pltpu.make_async_copy(
                out_hbm.at[pl.ds(0, 16)], stage_v.at[pl.ds(0, 16)], sem_sa
            ).wait()

        def drain_b():
            pltpu.make_async_copy(
                out_hbm.at[pl.ds(0, 16)], stage_v.at[pl.ds(16, 16)], sem_sb
            ).wait()

        def extract_one(t, _, buf):
            slot = state_s[0]
            fa = state_s[1]
            fb = state_s[2]
            v16 = cval_v[pl.ds(t, 16)]
            p16 = cpos_v[pl.ds(t, 16)]
            xs = jnp.full((16,), v16[0], jnp.int32)
            ps = p16[0]
            sl = slot & 15
            half = slot >> 4

            drained_a = (sl == 0) & (half == 0) & (fa > 0)
            drained_b = (sl == 0) & (half == 1) & (fb > 0)

            @pl.when(drained_a)
            def _():
                drain_a()
                state_s[1] = jnp.int32(0)

            @pl.when(drained_b)
            def _():
                drain_b()
                state_s[2] = jnp.int32(0)

            slotv = jnp.full((16,), slot & 31, jnp.int32)
            for gg in range(4):
                vals = plsc.load_gather(buf, [gg * 16 + lane, xs])
                plsc.store_scatter(stage_v, [slotv, gg * 16 + lane], vals)

            acc = spos_v[2, :]
            acc = jnp.where(lane == sl, ps, acc)
            flush = sl == 15
            spos_v[2, :] = jnp.where(flush, dump16, acc)

            @pl.when(flush & (half == 0))
            def _():
                spos_v[0, :] = acc
                pltpu.async_copy(
                    stage_v.at[pl.ds(0, 16)], out_hbm.at[spos_v.at[0]], sem_sa)
                state_s[1] = jnp.int32(1)

            @pl.when(flush & (half == 1))
            def _():
                spos_v[1, :] = acc
                pltpu.async_copy(
                    stage_v.at[pl.ds(16, 16)], out_hbm.at[spos_v.at[1]], sem_sb)
                state_s[2] = jnp.int32(1)

            state_s[0] = (slot + 1) & 31
            return 0

        def process(c, buf):
            xbase = jnp.where(c == _TAIL_C, _k(_TAIL_LO), c * _CCOLS)
            active = c <= _TAIL_C

            def pass_body(carry):
                done, _, _ = carry

                def rescan(q, rc):
                    r, mc = rc
                    v16 = mval_v[pl.ds(q * 16, 16)]
                    p16 = mpos_v[pl.ds(q * 16, 16)]
                    member = ((v16 >> 9) == c) & (q * 16 + lane < m)
                    mi = member.astype(jnp.int32)
                    order = r + plsc.cumsum(mi) - 1
                    take = member & (order >= done) & (order < done + _CAP)
                    plsc.store_compressed(
                        cval_v.at[pl.ds(mc, 16)], v16 - xbase, mask=take)
                    plsc.store_compressed(
                        cpos_v.at[pl.ds(mc, 16)], p16, mask=take)
                    return (r + jnp.sum(mi), mc + jnp.sum(take.astype(jnp.int32)))

                ng = jnp.where(active, n_groups + 2, jnp.int32(0))
                _, mc = lax.fori_loop(0, ng, rescan, (jnp.int32(0), jnp.int32(0)))
                lax.fori_loop(0, mc, lambda t, s: extract_one(t, s, buf), 0)
                return (done + mc, mc, jnp.int32(0))

            def pass_cond(carry):
                _, mc_last, first = carry
                return (first > 0) | (mc_last == _CAP)

            lax.while_loop(pass_cond, pass_body,
                           (jnp.int32(0), jnp.int32(0), jnp.int32(1)))

        state_s[0] = jnp.int32(0)
        state_s[1] = jnp.int32(0)
        state_s[2] = jnp.int32(0)
        spos_v[2, :] = dump16
        fire(wid, chunk_a, sem_ca)

        def chunk_pair(u, _):
            c_even = wid + u * 2 * _NW
            c_odd = c_even + _NW
            fire(c_odd, chunk_b, sem_cb)
            wait(c_even, chunk_a, sem_ca)
            process(c_even, chunk_a)
            fire(c_even + 2 * _NW, chunk_a, sem_ca)
            wait(c_odd, chunk_b, sem_cb)
            process(c_odd, chunk_b)
            return 0

        lax.fori_loop(0, _JMAX // 2, chunk_pair, 0)

        slot = state_s[0]
        half = slot >> 4

        @pl.when(state_s[1] > 0)
        def _():
            drain_a()

        @pl.when(state_s[2] > 0)
        def _():
            drain_b()

        acc = spos_v[2, :]

        @pl.when(half == 0)
        def _():
            spos_v[0, :] = acc
            pltpu.async_copy(
                stage_v.at[pl.ds(0, 16)], out_hbm.at[spos_v.at[0]], sem_sa
            ).wait()

        @pl.when(half == 1)
        def _():
            spos_v[1, :] = acc
            pltpu.async_copy(
                stage_v.at[pl.ds(16, 16)], out_hbm.at[spos_v.at[1]], sem_sb
            ).wait()

    return body


def kernel(g, h, r, norm, W):
    idx = jnp.squeeze(h).astype(jnp.int32)
    table = W.T
    tail = W[_TAIL_LO:, :].T
    out = _build()(idx, table, tail)
    return out[:_B, :_D]


# speedup vs baseline: 4.0598x; 1.0021x over previous
import functools

import jax
import jax.numpy as jnp
from jax import lax
from jax.experimental import pallas as pl
from jax.experimental.pallas import tpu as pltpu
from jax.experimental.pallas import tpu_sc as plsc

_INFO = plsc.get_sparse_core_info()
_NC = _INFO.num_cores
_NS = _INFO.num_subcores
_NW = _NC * _NS

_N = 1000000
_B = 16384
_D = 64
_CCOLS = 512
_NCHUNK = 1954
_TAIL_C = _NCHUNK - 1
_TAIL_LO = _N - 128
_JMAX = 62
_CAP = 2048
_DUMP = _B


def _k(c):
    return jnp.int32(c)


@functools.lru_cache(maxsize=None)
def _build():
    mesh = plsc.VectorSubcoreMesh(core_axis_name="c", subcore_axis_name="s")

    @functools.partial(
        pl.kernel,
        mesh=mesh,
        out_type=jax.ShapeDtypeStruct((_B + 16, 128), jnp.float32),
        scratch_types=[
            pltpu.VMEM((_B,), jnp.int32),
            pltpu.VMEM((_B + 64,), jnp.int32),
            pltpu.VMEM((_B + 64,), jnp.int32),
            pltpu.VMEM((_CAP + 16,), jnp.int32),
            pltpu.VMEM((_CAP + 16,), jnp.int32),
            pltpu.VMEM((_D, _CCOLS), jnp.float32),
            pltpu.VMEM((_D, _CCOLS), jnp.float32),
            pltpu.VMEM((32, 128), jnp.float32),
            pltpu.VMEM((3, 16), jnp.int32),
            pltpu.SMEM((8,), jnp.int32),
            pltpu.SemaphoreType.DMA,
            pltpu.SemaphoreType.DMA,
            pltpu.SemaphoreType.DMA,
            pltpu.SemaphoreType.DMA,
        ],
        compiler_params=pltpu.CompilerParams(needs_layout_passes=False),
    )
    def body(idx_hbm, table_hbm, tail_hbm, out_hbm, idx_v, mval_v, mpos_v,
             cval_v, cpos_v, chunk_a, chunk_b, stage_v, spos_v, state_s,
             sem_ca, sem_cb, sem_sa, sem_sb):
        wid = lax.axis_index("s") * _NC + lax.axis_index("c")
        lane = lax.iota(jnp.int32, 16)
        dump16 = _DUMP + lane

        pltpu.sync_copy(idx_hbm, idx_v)

        def scan_body(q, m):
            v = idx_v[pl.ds(q * 16, 16)]
            mask = ((v >> 9) & (_NW - 1)) == wid
            bpos = q * 16 + lane
            plsc.store_compressed(mval_v.at[pl.ds(m, 16)], v, mask=mask)
            plsc.store_compressed(mpos_v.at[pl.ds(m, 16)], bpos, mask=mask)
            return m + jnp.sum(mask.astype(jnp.int32))

        m = lax.fori_loop(0, _B // 16, scan_body, jnp.int32(0))
        n_groups = (m + 15) >> 4

        def fire(c, buf, sem):
            @pl.when(c <= _TAIL_C - 1)
            def _():
                pltpu.async_copy(
                    table_hbm.at[:, pl.ds(c * _CCOLS, _CCOLS)], buf, sem)

            @pl.when(c == _TAIL_C)
            def _():
                pltpu.async_copy(tail_hbm, buf.at[:, pl.ds(0, 128)], sem)

        def wait(c, buf, sem):
            @pl.when(c <= _TAIL_C - 1)
            def _():
                pltpu.make_async_copy(
                    table_hbm.at[:, pl.ds(c * _CCOLS, _CCOLS)], buf, sem
                ).wait()

            @pl.when(c == _TAIL_C)
            def _():
                pltpu.make_async_copy(
                    tail_hbm, buf.at[:, pl.ds(0, 128)], sem).wait()

        def drain_a():
            pltpu.make_async_copy(
                out_hbm.at[pl.ds(0, 16)], stage_v.at[pl.ds(0, 16)], sem_sa
            ).wait()

        def drain_b():
            pltpu.make_async_copy(
                out_hbm.at[pl.ds(0, 16)], stage_v.at[pl.ds(16, 16)], sem_sb
            ).wait()

        def extract_one(t, _, buf):
            slot = state_s[0]
            fa = state_s[1]
            fb = state_s[2]
            v16 = cval_v[pl.ds(t, 16)]
            p16 = cpos_v[pl.ds(t, 16)]
            xs = jnp.full((16,), v16[0], jnp.int32)
            ps = p16[0]
            sl = slot & 15
            half = slot >> 4

            drained_a = (sl == 0) & (half == 0) & (fa > 0)
            drained_b = (sl == 0) & (half == 1) & (fb > 0)

            @pl.when(drained_a)
            def _():
                drain_a()
                state_s[1] = jnp.int32(0)

            @pl.when(drained_b)
            def _():
                drain_b()
                state_s[2] = jnp.int32(0)

            slotv = jnp.full((16,), slot & 31, jnp.int32)
            for gg in range(4):
                vals = plsc.load_gather(buf, [gg * 16 + lane, xs])
                plsc.store_scatter(stage_v, [slotv, gg * 16 + lane], vals)

            acc = spos_v[2, :]
            acc = jnp.where(lane == sl, ps, acc)
            flush = sl == 15
            spos_v[2, :] = jnp.where(flush, dump16, acc)

            @pl.when(flush & (half == 0))
            def _():
                spos_v[0, :] = acc
                pltpu.async_copy(
                    stage_v.at[pl.ds(0, 16)], out_hbm.at[spos_v.at[0]], sem_sa)
                state_s[1] = jnp.int32(1)

            @pl.when(flush & (half == 1))
            def _():
                spos_v[1, :] = acc
                pltpu.async_copy(
                    stage_v.at[pl.ds(16, 16)], out_hbm.at[spos_v.at[1]], sem_sb)
                state_s[2] = jnp.int32(1)

            state_s[0] = (slot + 1) & 31
            return 0

        def process(c, buf):
            xbase = jnp.where(c == _TAIL_C, _k(_TAIL_LO), c * _CCOLS)
            active = c <= _TAIL_C

            def pass_body(carry):
                done, _, _ = carry

                def rescan(q, rc):
                    r, mc = rc
                    v16 = mval_v[pl.ds(q * 16, 16)]
                    p16 = mpos_v[pl.ds(q * 16, 16)]
                    member = ((v16 >> 9) == c) & (q * 16 + lane < m)
                    mi = member.astype(jnp.int32)
                    order = r + plsc.cumsum(mi) - 1
                    take = member & (order >= done) & (order < done + _CAP)
                    plsc.store_compressed(
                        cval_v.at[pl.ds(mc, 16)], v16 - xbase, mask=take)
                    plsc.store_compressed(
                        cpos_v.at[pl.ds(mc, 16)], p16, mask=take)
                    return (r + jnp.sum(mi), mc + jnp.sum(take.astype(jnp.int32)))

                ng = jnp.where(active, n_groups + 2, jnp.int32(0))
                _, mc = lax.fori_loop(0, ng, rescan, (jnp.int32(0), jnp.int32(0)))
                lax.fori_loop(0, mc, lambda t, s: extract_one(t, s, buf), 0)
                return (done + mc, mc, jnp.int32(0))

            def pass_cond(carry):
                _, mc_last, first = carry
                return (first > 0) | (mc_last == _CAP)

            lax.while_loop(pass_cond, pass_body,
                           (jnp.int32(0), jnp.int32(0), jnp.int32(1)))

        state_s[0] = jnp.int32(0)
        state_s[1] = jnp.int32(0)
        state_s[2] = jnp.int32(0)
        spos_v[2, :] = dump16
        fire(wid, chunk_a, sem_ca)

        def chunk_pair(u, _):
            c_even = wid + u * 2 * _NW
            c_odd = c_even + _NW
            fire(c_odd, chunk_b, sem_cb)
            wait(c_even, chunk_a, sem_ca)
            process(c_even, chunk_a)
            fire(c_even + 2 * _NW, chunk_a, sem_ca)
            wait(c_odd, chunk_b, sem_cb)
            process(c_odd, chunk_b)
            return 0

        lax.fori_loop(0, _JMAX // 2, chunk_pair, 0)

        slot = state_s[0]
        half = slot >> 4

        @pl.when(state_s[1] > 0)
        def _():
            drain_a()

        @pl.when(state_s[2] > 0)
        def _():
            drain_b()

        acc = spos_v[2, :]

        @pl.when(half == 0)
        def _():
            spos_v[0, :] = acc
            pltpu.async_copy(
                stage_v.at[pl.ds(0, 16)], out_hbm.at[spos_v.at[0]], sem_sa
            ).wait()

        @pl.when(half == 1)
        def _():
            spos_v[1, :] = acc
            pltpu.async_copy(
                stage_v.at[pl.ds(16, 16)], out_hbm.at[spos_v.at[1]], sem_sb
            ).wait()

    return body


def kernel(g, h, r, norm, W):
    idx = jnp.squeeze(h).astype(jnp.int32)
    table = W.T
    tail = W[_TAIL_LO:, :].T
    out = _build()(idx, table, tail)
    return out[:_B, :_D]
